# initial kernel scaffold (unmeasured)
import jax
import jax.numpy as jnp
from jax import lax
from jax.experimental import pallas as pl
from jax.experimental.pallas import tpu as pltpu


def kernel(
    x,
):
    def body(*refs):
        pass

    out_shape = jax.ShapeDtypeStruct(..., jnp.float32)
    return pl.pallas_call(body, out_shape=out_shape)(...)


# baseline (device time: 36852 ns/iter reference)
import jax
import jax.numpy as jnp
from jax import lax
from jax.experimental import pallas as pl
from jax.experimental.pallas import tpu as pltpu

W = 16
M = 256
LOG_M = 8


def _ce_stage(x, gi_base, j, k):
    r = x.shape[0]
    ii = lax.broadcasted_iota(jnp.int32, x.shape, 0) + gi_base
    lower = (ii & j) == 0
    asc = (ii & k) == 0
    take_min = lower == asc
    up = pltpu.roll(x, r - j, 0)
    dn = pltpu.roll(x, j, 0)
    partner = jnp.where(lower, up, dn)
    return jnp.where(take_min, jnp.minimum(x, partner), jnp.maximum(x, partner))


def kernel(x):
    m, n = x.shape
    assert m == M

    def body(x_ref, out_ref, cur_ref, recv_ref, send_sems, recv_sems):
        my = lax.axis_index("i")
        gi = my * M

        barrier = pltpu.get_barrier_semaphore()
        for lsp in range(4):
            pl.semaphore_signal(
                barrier, inc=1,
                device_id=(my ^ (1 << lsp),),
                device_id_type=pl.DeviceIdType.MESH,
            )
        pl.semaphore_wait(barrier, 4)

        v = x_ref[...].astype(jnp.bfloat16)
        for lk in range(1, LOG_M + 1):
            k = 1 << lk
            for lj in range(lk - 1, -1, -1):
                v = _ce_stage(v, gi, 1 << lj, k)
        cur_ref[...] = v

        t = 0
        for lp in range(1, 5):
            p_blk = 1 << lp
            k = p_blk * M
            for lsp in range(lp - 1, -1, -1):
                sp = 1 << lsp
                rdma = pltpu.make_async_remote_copy(
                    src_ref=cur_ref,
                    dst_ref=recv_ref.at[t],
                    send_sem=send_sems.at[t],
                    recv_sem=recv_sems.at[t],
                    device_id=(my ^ sp,),
                    device_id_type=pl.DeviceIdType.MESH,
                )
                rdma.start()
                rdma.wait()
                mine = cur_ref[...]
                other = recv_ref[t]
                ii = lax.broadcasted_iota(jnp.int32, mine.shape, 0) + gi
                take_min = ((ii & (sp * M)) == 0) == ((ii & k) == 0)
                cur_ref[...] = jnp.where(
                    take_min,
                    jnp.minimum(mine, other),
                    jnp.maximum(mine, other),
                )
                t += 1
            v = cur_ref[...]
            for lj in range(LOG_M - 1, -1, -1):
                v = _ce_stage(v, gi, 1 << lj, k)
            cur_ref[...] = v

        out_ref[...] = cur_ref[...].astype(jnp.float32)

    n_cross = 10
    return pl.pallas_call(
        body,
        out_shape=jax.ShapeDtypeStruct((M, n), jnp.float32),
        in_specs=[pl.BlockSpec(memory_space=pltpu.VMEM)],
        out_specs=pl.BlockSpec(memory_space=pltpu.VMEM),
        scratch_shapes=[
            pltpu.VMEM((M, n), jnp.bfloat16),
            pltpu.VMEM((n_cross, M, n), jnp.bfloat16),
            pltpu.SemaphoreType.DMA((n_cross,)),
            pltpu.SemaphoreType.DMA((n_cross,)),
        ],
        compiler_params=pltpu.CompilerParams(collective_id=7),
    )(x)


# device time: 36844 ns/iter; 1.0002x vs baseline; 1.0002x over previous
import jax
import jax.numpy as jnp
from jax import lax
from jax.experimental import pallas as pl
from jax.experimental.pallas import tpu as pltpu

W = 16
M = 256
LOG_M = 8


def _ce_stage(x, gi_base, j, k):
    r = x.shape[0]
    ii = lax.broadcasted_iota(jnp.int32, x.shape, 0) + gi_base
    lower = (ii & j) == 0
    asc = (ii & k) == 0
    take_min = lower == asc
    up = pltpu.roll(x, r - j, 0)
    dn = pltpu.roll(x, j, 0)
    partner = jnp.where(lower, up, dn)
    return jnp.where(take_min, jnp.minimum(x, partner), jnp.maximum(x, partner))


def kernel(x):
    m, n = x.shape
    assert m == M

    def body(x_ref, out_ref, cur_ref, recv_ref, send_sems, recv_sems):
        my = lax.axis_index("i")
        gi = my * M

        v = x_ref[...].astype(jnp.bfloat16)
        for lk in range(1, LOG_M + 1):
            k = 1 << lk
            for lj in range(lk - 1, -1, -1):
                v = _ce_stage(v, gi, 1 << lj, k)
        cur_ref[...] = v

        barrier = pltpu.get_barrier_semaphore()
        for lsp in range(4):
            pl.semaphore_signal(
                barrier, inc=1,
                device_id=(my ^ (1 << lsp),),
                device_id_type=pl.DeviceIdType.MESH,
            )
        pl.semaphore_wait(barrier, 4)

        t = 0
        for lp in range(1, 5):
            p_blk = 1 << lp
            k = p_blk * M
            for lsp in range(lp - 1, -1, -1):
                sp = 1 << lsp
                rdma = pltpu.make_async_remote_copy(
                    src_ref=cur_ref,
                    dst_ref=recv_ref.at[t],
                    send_sem=send_sems.at[t],
                    recv_sem=recv_sems.at[t],
                    device_id=(my ^ sp,),
                    device_id_type=pl.DeviceIdType.MESH,
                )
                rdma.start()
                rdma.wait()
                mine = cur_ref[...]
                other = recv_ref[t]
                ii = lax.broadcasted_iota(jnp.int32, mine.shape, 0) + gi
                take_min = ((ii & (sp * M)) == 0) == ((ii & k) == 0)
                cur_ref[...] = jnp.where(
                    take_min,
                    jnp.minimum(mine, other),
                    jnp.maximum(mine, other),
                )
                t += 1
            v = cur_ref[...]
            for lj in range(LOG_M - 1, -1, -1):
                v = _ce_stage(v, gi, 1 << lj, k)
            cur_ref[...] = v

        out_ref[...] = cur_ref[...].astype(jnp.float32)

    n_cross = 10
    return pl.pallas_call(
        body,
        out_shape=jax.ShapeDtypeStruct((M, n), jnp.float32),
        in_specs=[pl.BlockSpec(memory_space=pltpu.VMEM)],
        out_specs=pl.BlockSpec(memory_space=pltpu.VMEM),
        scratch_shapes=[
            pltpu.VMEM((M, n), jnp.bfloat16),
            pltpu.VMEM((n_cross, M, n), jnp.bfloat16),
            pltpu.SemaphoreType.DMA((n_cross,)),
            pltpu.SemaphoreType.DMA((n_cross,)),
        ],
        compiler_params=pltpu.CompilerParams(collective_id=7),
    )(x)


# device time: 32222 ns/iter; 1.1437x vs baseline; 1.1434x over previous
import jax
import jax.numpy as jnp
from jax import lax
from jax.experimental import pallas as pl
from jax.experimental.pallas import tpu as pltpu

W = 16
M = 256
LOG_M = 8

_ROUNDS = [
    ("single", 1, 2, True),
    ("pair", 1, 4, True),
    ("pair", 2, 8, False),
    ("single", 1, 8, True),
    ("pair", 4, 16, False),
    ("pair", 1, 16, True),
]
_N_SLOTS = sum(3 if r[0] == "pair" else 1 for r in _ROUNDS)
_PEERS = sorted({d * q for k, q, _, _ in _ROUNDS
                 for d in ((1, 2, 3) if k == "pair" else (1,))})


def _ce_stage(x, gi_base, j, k):
    r = x.shape[0]
    ii = lax.broadcasted_iota(jnp.int32, x.shape, 0) + gi_base
    lower = (ii & j) == 0
    asc = (ii & k) == 0
    take_min = lower == asc
    up = pltpu.roll(x, r - j, 0)
    dn = pltpu.roll(x, j, 0)
    partner = jnp.where(lower, up, dn)
    return jnp.where(take_min, jnp.minimum(x, partner), jnp.maximum(x, partner))


def _mm(ii, sp, p_blk, a, b):
    tm = ((ii & (sp * M)) == 0) == ((ii & (p_blk * M)) == 0)
    return jnp.where(tm, jnp.minimum(a, b), jnp.maximum(a, b))


def kernel(x):
    m, n = x.shape
    assert m == M

    def body(x_ref, out_ref, cur_ref, recv_ref, send_sems, recv_sems):
        my = lax.axis_index("i")
        gi = my * M

        v = x_ref[...].astype(jnp.bfloat16)
        for lk in range(1, LOG_M + 1):
            k = 1 << lk
            for lj in range(lk - 1, -1, -1):
                v = _ce_stage(v, gi, 1 << lj, k)
        cur_ref[...] = v

        barrier = pltpu.get_barrier_semaphore()
        for d in _PEERS:
            pl.semaphore_signal(
                barrier, inc=1,
                device_id=(my ^ d,),
                device_id_type=pl.DeviceIdType.MESH,
            )
        pl.semaphore_wait(barrier, len(_PEERS))

        base = 0
        for kind, q, p_blk, phase_end in _ROUNDS:
            k = p_blk * M
            deltas = (q, 2 * q, 3 * q) if kind == "pair" else (q,)
            rdmas = []
            for idx, d in enumerate(deltas):
                rdma = pltpu.make_async_remote_copy(
                    src_ref=cur_ref,
                    dst_ref=recv_ref.at[base + idx],
                    send_sem=send_sems.at[base + idx],
                    recv_sem=recv_sems.at[base + idx],
                    device_id=(my ^ d,),
                    device_id_type=pl.DeviceIdType.MESH,
                )
                rdma.start()
                rdmas.append(rdma)
            for rdma in rdmas:
                rdma.wait()

            ii = lax.broadcasted_iota(jnp.int32, (M, n), 0) + gi
            if kind == "single":
                cur_ref[...] = _mm(ii, q, p_blk, cur_ref[...], recv_ref[base])
            else:
                s0 = cur_ref[...]
                s1 = recv_ref[base]
                s2 = recv_ref[base + 1]
                s3 = recv_ref[base + 2]
                a_my = _mm(ii, 2 * q, p_blk, s0, s2)
                a_q = _mm(ii ^ (q * M), 2 * q, p_blk, s1, s3)
                cur_ref[...] = _mm(ii, q, p_blk, a_my, a_q)
            base += len(deltas)

            if phase_end:
                v = cur_ref[...]
                for lj in range(LOG_M - 1, -1, -1):
                    v = _ce_stage(v, gi, 1 << lj, k)
                cur_ref[...] = v

        out_ref[...] = cur_ref[...].astype(jnp.float32)

    return pl.pallas_call(
        body,
        out_shape=jax.ShapeDtypeStruct((M, n), jnp.float32),
        in_specs=[pl.BlockSpec(memory_space=pltpu.VMEM)],
        out_specs=pl.BlockSpec(memory_space=pltpu.VMEM),
        scratch_shapes=[
            pltpu.VMEM((M, n), jnp.bfloat16),
            pltpu.VMEM((_N_SLOTS, M, n), jnp.bfloat16),
            pltpu.SemaphoreType.DMA((_N_SLOTS,)),
            pltpu.SemaphoreType.DMA((_N_SLOTS,)),
        ],
        compiler_params=pltpu.CompilerParams(collective_id=7),
    )(x)
